# Initial kernel scaffold; baseline (speedup 1.0000x reference)
import jax
import jax.numpy as jnp
from jax.experimental import pallas as pl


def kernel(user_ids, item_ids, user_table, item_table, W, b):
    raise NotImplementedError("write your pallas kernel here")


# speedup vs baseline: 1.1674x; 1.1674x over previous
import functools
import jax
import jax.numpy as jnp
from jax import lax
from jax.experimental import pallas as pl
from jax.experimental.pallas import tpu as pltpu
from jax.experimental.pallas import tpu_sc as plsc

EMBED_DIM = 128
LANES = 16
D_CHUNKS = EMBED_DIM // LANES
NUM_CORES = 2
NUM_SUBCORES = 16
NUM_WORKERS = NUM_CORES * NUM_SUBCORES
ROW_CHUNK = 128


def _make_gmf(batch):
    b_per_w = batch // NUM_WORKERS
    n_chunks = b_per_w // ROW_CHUNK
    mesh = plsc.VectorSubcoreMesh(core_axis_name="c", subcore_axis_name="s")

    @functools.partial(
        pl.kernel,
        mesh=mesh,
        compiler_params=pltpu.CompilerParams(needs_layout_passes=False),
        out_type=jax.ShapeDtypeStruct((NUM_WORKERS, b_per_w), jnp.float32),
        scratch_types=[
            pltpu.VMEM((n_chunks, ROW_CHUNK), jnp.int32),
            pltpu.VMEM((n_chunks, ROW_CHUNK), jnp.int32),
            pltpu.VMEM((ROW_CHUNK, EMBED_DIM), jnp.float32),
            pltpu.VMEM((ROW_CHUNK, EMBED_DIM), jnp.float32),
            pltpu.VMEM((D_CHUNKS, LANES), jnp.float32),
            pltpu.VMEM((LANES,), jnp.float32),
            pltpu.VMEM((b_per_w,), jnp.float32),
            pltpu.SemaphoreType.DMA,
            pltpu.SemaphoreType.DMA,
        ],
    )
    def gmf(uid_hbm, iid_hbm, ut_hbm, it_hbm, w_hbm, bias_hbm, out_hbm,
            uidx_v, iidx_v, urows_v, irows_v, w_v, bias_v, out_v,
            sem_u, sem_i):
        wid = lax.axis_index("s") * NUM_CORES + lax.axis_index("c")
        pltpu.sync_copy(uid_hbm.at[wid], uidx_v)
        pltpu.sync_copy(iid_hbm.at[wid], iidx_v)
        pltpu.sync_copy(w_hbm, w_v)
        pltpu.sync_copy(bias_hbm, bias_v)
        bias_vec = bias_v[...]
        w_vecs = [w_v[j] for j in range(D_CHUNKS)]
        lane_iota = lax.iota(jnp.int32, LANES)

        for c in range(n_chunks):
            pltpu.async_copy(ut_hbm.at[uidx_v.at[c]], urows_v, sem_u).wait()
            pltpu.async_copy(it_hbm.at[iidx_v.at[c]], irows_v, sem_i).wait()

            def grp_body(g, _, c=c):
                def row_body(r, vec):
                    rr = g * LANES + r
                    acc = (urows_v[rr, pl.ds(0, LANES)]
                           * irows_v[rr, pl.ds(0, LANES)] * w_vecs[0])
                    for j in range(1, D_CHUNKS):
                        acc = acc + (urows_v[rr, pl.ds(j * LANES, LANES)]
                                     * irows_v[rr, pl.ds(j * LANES, LANES)]
                                     * w_vecs[j])
                    return jnp.where(lane_iota == r, jnp.sum(acc), vec)

                vec = lax.fori_loop(0, LANES, row_body,
                                    jnp.zeros((LANES,), jnp.float32))
                off = pl.multiple_of(c * ROW_CHUNK + g * LANES, LANES)
                out_v[pl.ds(off, LANES)] = vec + bias_vec
                return 0

            lax.fori_loop(0, ROW_CHUNK // LANES, grp_body, 0)

        pltpu.sync_copy(out_v, out_hbm.at[wid])

    return gmf


_gmf_cached = {}


def kernel(user_ids, item_ids, user_table, item_table, W, b):
    batch = user_ids.shape[0]
    if batch not in _gmf_cached:
        _gmf_cached[batch] = _make_gmf(batch)
    gmf = _gmf_cached[batch]
    b_per_w = batch // NUM_WORKERS
    n_chunks = b_per_w // ROW_CHUNK
    uid = user_ids.astype(jnp.int32).reshape(NUM_WORKERS, n_chunks, ROW_CHUNK)
    iid = item_ids.astype(jnp.int32).reshape(NUM_WORKERS, n_chunks, ROW_CHUNK)
    w = W.reshape(D_CHUNKS, LANES)
    b16 = jnp.broadcast_to(b.reshape(()), (LANES,))
    out = gmf(uid, iid, user_table, item_table, w, b16)
    return out.reshape(batch)
